# baseline (speedup 1.0000x reference)
import jax
import jax.numpy as jnp
from jax.experimental import pallas as pl
from jax.experimental.pallas import tpu as pltpu

_I = 3
_J = 700
_ROWS = 16384
_COLS = 1024
_CHUNKS = (1024, 4096, 4096, 4096, 2048, 1024)
_OFFS = tuple(sum(_CHUNKS[:i]) for i in range(len(_CHUNKS)))
_N = len(_CHUNKS)
_BMAX = max(_CHUNKS)
_NBUF = 3


def _body(theta_ref, x_ref, o_ref, bufs, in_sems, out_sems):
    t = theta_ref[0]
    c = jnp.cos(2.0 * t)
    s = jnp.sin(2.0 * t)

    def in_cp(i):
        return pltpu.make_async_copy(
            x_ref.at[pl.ds(_OFFS[i], _CHUNKS[i]), :],
            bufs.at[i % _NBUF, pl.ds(0, _CHUNKS[i]), :], in_sems.at[i])

    def out_cp(i):
        return pltpu.make_async_copy(
            bufs.at[i % _NBUF, pl.ds(0, _CHUNKS[i]), :],
            o_ref.at[pl.ds(_OFFS[i], _CHUNKS[i]), :], out_sems.at[i])

    for i in range(_NBUF):
        in_cp(i).start()
    for i in range(_N):
        b = i % _NBUF
        r = _CHUNKS[i]
        in_cp(i).wait()
        xi = bufs[b, 0:r, _I:_I + 1]
        xj = bufs[b, 0:r, _J:_J + 1]
        bufs[b, 0:r, _I:_I + 1] = xi * c + xj * s
        bufs[b, 0:r, _J:_J + 1] = xi * s - xj * c
        out_cp(i).start()
        k = i + _NBUF
        if k < _N:
            out_cp(i).wait()
            in_cp(k).start()
    for i in range(_N - _NBUF, _N):
        out_cp(i).wait()


def kernel(x, theta):
    theta_arr = jnp.reshape(theta, (1,)).astype(jnp.float32)
    return pl.pallas_call(
        _body,
        in_specs=[
            pl.BlockSpec(memory_space=pltpu.SMEM),
            pl.BlockSpec(memory_space=pl.ANY),
        ],
        out_specs=pl.BlockSpec(memory_space=pl.ANY),
        out_shape=jax.ShapeDtypeStruct((_ROWS, _COLS), jnp.float32),
        scratch_shapes=[
            pltpu.VMEM((_NBUF, _BMAX, _COLS), jnp.float32),
            pltpu.SemaphoreType.DMA((_N,)),
            pltpu.SemaphoreType.DMA((_N,)),
        ],
    )(theta_arr, x)


# speedup vs baseline: 1.0496x; 1.0496x over previous
import jax
import jax.numpy as jnp
from jax.experimental import pallas as pl
from jax.experimental.pallas import tpu as pltpu

_I = 3
_J = 700
_ROWS = 16384
_COLS = 1024
_CHUNKS = (1024, 4096, 4096, 4096, 2048, 1024)
_OFFS = tuple(sum(_CHUNKS[:i]) for i in range(len(_CHUNKS)))
_N = len(_CHUNKS)
_BUF = (3, 0, 1, 2, 0, 3)
_BUFSHAPES = (4096, 4096, 4096, 1024)
_PREV = tuple(
    max((j for j in range(i) if _BUF[j] == _BUF[i]), default=None)
    for i in range(_N))


def _body(theta_ref, x_ref, o_ref, b0, b1, b2, b3, in_sems, out_sems):
    bufs = (b0, b1, b2, b3)
    t = theta_ref[0]
    c = jnp.cos(2.0 * t)
    s = jnp.sin(2.0 * t)

    def in_cp(i):
        return pltpu.make_async_copy(
            x_ref.at[pl.ds(_OFFS[i], _CHUNKS[i]), :],
            bufs[_BUF[i]].at[pl.ds(0, _CHUNKS[i]), :], in_sems.at[i])

    def out_cp(i):
        return pltpu.make_async_copy(
            bufs[_BUF[i]].at[pl.ds(0, _CHUNKS[i]), :],
            o_ref.at[pl.ds(_OFFS[i], _CHUNKS[i]), :], out_sems.at[i])

    for i in range(_N):
        if _PREV[i] is None:
            in_cp(i).start()
    for i in range(_N):
        buf = bufs[_BUF[i]]
        r = _CHUNKS[i]
        in_cp(i).wait()
        xi = buf[0:r, _I:_I + 1]
        xj = buf[0:r, _J:_J + 1]
        buf[0:r, _I:_I + 1] = xi * c + xj * s
        buf[0:r, _J:_J + 1] = xi * s - xj * c
        out_cp(i).start()
        for k in range(i + 1, _N):
            if _PREV[k] == i:
                out_cp(i).wait()
                in_cp(k).start()
    for i in range(_N):
        if all(_PREV[k] != i for k in range(i + 1, _N)):
            out_cp(i).wait()


def kernel(x, theta):
    theta_arr = jnp.reshape(theta, (1,)).astype(jnp.float32)
    return pl.pallas_call(
        _body,
        in_specs=[
            pl.BlockSpec(memory_space=pltpu.SMEM),
            pl.BlockSpec(memory_space=pl.ANY),
        ],
        out_specs=pl.BlockSpec(memory_space=pl.ANY),
        out_shape=jax.ShapeDtypeStruct((_ROWS, _COLS), jnp.float32),
        scratch_shapes=[
            pltpu.VMEM((_BUFSHAPES[0], _COLS), jnp.float32),
            pltpu.VMEM((_BUFSHAPES[1], _COLS), jnp.float32),
            pltpu.VMEM((_BUFSHAPES[2], _COLS), jnp.float32),
            pltpu.VMEM((_BUFSHAPES[3], _COLS), jnp.float32),
            pltpu.SemaphoreType.DMA((_N,)),
            pltpu.SemaphoreType.DMA((_N,)),
        ],
    )(theta_arr, x)
